# Initial kernel scaffold; baseline (speedup 1.0000x reference)
import jax
import jax.numpy as jnp
from jax.experimental import pallas as pl


def kernel(x, multihop_edge_index, distance, edge_attr, eps, W1, b1, gamma, beta, W2, b2):
    raise NotImplementedError("write your pallas kernel here")


# speedup vs baseline: 15.1690x; 15.1690x over previous
import functools

import jax
import jax.numpy as jnp
from jax import lax
from jax.experimental import pallas as pl
from jax.experimental.pallas import tpu as pltpu
from jax.experimental.pallas import tpu_sc as plsc

N = 10000
E = 320000
DIM = 128
K = 4

NUM_CORES = 2
NUM_SUBCORES = 16
NUM_TILES = NUM_CORES * NUM_SUBCORES
EDGES_PER_TILE = E // NUM_TILES
BATCH = 80
NUM_BLOCKS = EDGES_PER_TILE // BATCH
ROW_CHUNK = 80
NUM_ROW_CHUNKS = N // ROW_CHUNK


def _build_tables_body(x_ref, eps_ref, out_ref):
    xb = x_ref[...]
    rb = jnp.maximum(xb, 0.0)
    out_ref[0] = xb
    for j in range(1, K):
        out_ref[j] = rb * (1.0 + eps_ref[j + 1])[None, :]


def _build_tables(x, eps):
    blk = 1000
    return pl.pallas_call(
        _build_tables_body,
        grid=(N // blk,),
        in_specs=[
            pl.BlockSpec((blk, DIM), lambda i: (i, 0)),
            pl.BlockSpec((K + 1, DIM), lambda i: (0, 0)),
        ],
        out_specs=pl.BlockSpec((K, blk, DIM), lambda i: (0, i, 0)),
        out_shape=jax.ShapeDtypeStruct((K, N, DIM), jnp.float32),
    )(x, eps)


def _sc_edges_body(u_hbm, src_hbm, dst_hbm, dist_hbm, attr_hbm, eps_hbm,
                   out_hbm,
                   srcv_a, srcv_b, distv_a, distv_b, dstv_a, dstv_b,
                   gidx_a, gidx_b, rows_a, rows_b, attrv_a, attrv_b,
                   dchk_a, dchk_b, epsv, s_ref, acc,
                   isem_a, isem_b, dsem_a, dsem_b, gsem_a, gsem_b,
                   asem_a, asem_b, ssem_a, ssem_b):
    core = lax.axis_index("c")
    sub = lax.axis_index("s")
    tile = core * NUM_SUBCORES + sub

    pltpu.sync_copy(eps_hbm, epsv)
    for j in range(DIM // 16):
        s_ref[pl.ds(16 * j, 16)] = epsv[pl.ds(DIM + 16 * j, 16)] + 1.0

    zeros16 = jnp.zeros((16,), jnp.float32)

    def _zero_row(r, _):
        for j in range(DIM // 16):
            rows_a[r, pl.ds(16 * j, 16)] = zeros16
        return 0

    lax.fori_loop(0, ROW_CHUNK, _zero_row, 0)

    def _zero_chunk(t, _):
        c = sub + NUM_SUBCORES * t

        @pl.when(c < NUM_ROW_CHUNKS)
        def _():
            r0 = pl.multiple_of(c * ROW_CHUNK, 8)
            pltpu.sync_copy(rows_a, acc.at[pl.ds(r0, ROW_CHUNK)])
        return 0

    lax.fori_loop(0, pl.cdiv(NUM_ROW_CHUNKS, NUM_SUBCORES), _zero_chunk, 0)
    plsc.subcore_barrier()

    def _off(j):
        return pl.multiple_of(tile * BATCH + j * (NUM_TILES * BATCH), 8)

    bufs = (
        (srcv_a, distv_a, dstv_a, gidx_a, rows_a, attrv_a, dchk_a,
         isem_a, dsem_a, gsem_a, asem_a, ssem_a),
        (srcv_b, distv_b, dstv_b, gidx_b, rows_b, attrv_b, dchk_b,
         isem_b, dsem_b, gsem_b, asem_b, ssem_b),
    )

    def idx_start(j, p):
        srcv, distv = bufs[p][0], bufs[p][1]
        off = _off(j)
        pltpu.async_copy(src_hbm.at[pl.ds(off, BATCH)], srcv, bufs[p][7])
        pltpu.async_copy(dist_hbm.at[pl.ds(off, BATCH)], distv, bufs[p][7])

    def dst_start(j, p):
        off = _off(j)
        pltpu.async_copy(dst_hbm.at[pl.ds(off, BATCH)], bufs[p][2],
                         bufs[p][8])

    def launch(j, p):
        srcv, distv, _, gidx, rows, attrv, dchk = bufs[p][:7]
        off = _off(j)
        pltpu.make_async_copy(src_hbm.at[pl.ds(off, BATCH)], srcv,
                              bufs[p][7]).wait()
        pltpu.make_async_copy(dist_hbm.at[pl.ds(off, BATCH)], distv,
                              bufs[p][7]).wait()
        for i in range(BATCH // 16):
            sl = pl.ds(16 * i, 16)
            gidx[sl] = (distv[sl] - 1) * N + srcv[sl]
        dchk[pl.ds(0, 16)] = distv[pl.ds(0, 16)]
        dchk[pl.ds(16, 16)] = distv[pl.ds(BATCH - 16, 16)]
        d1 = jnp.min(distv[pl.ds(0, 16)])

        @pl.when(d1 == 1)
        def _():
            pltpu.async_copy(attr_hbm.at[pl.ds(off, BATCH)], attrv,
                             bufs[p][10])

        pltpu.async_copy(u_hbm.at[gidx], rows, bufs[p][9])

    def process(j, p):
        _, _, dstv, gidx, rows, attrv, dchk = bufs[p][:7]
        off = _off(j)
        pltpu.make_async_copy(u_hbm.at[gidx], rows, bufs[p][9]).wait()
        d1 = jnp.min(dchk[pl.ds(0, 16)])
        dl = jnp.max(dchk[pl.ds(16, 16)])

        @pl.when(d1 == 1)
        def _():
            pltpu.make_async_copy(attr_hbm.at[pl.ds(off, BATCH)], attrv,
                                  bufs[p][10]).wait()

        @pl.when(dl == 1)
        def _pure_hop1():
            def _edge(e, _):
                for jj in range(DIM // 16):
                    sl = pl.ds(16 * jj, 16)
                    rows[e, sl] = (jnp.maximum(rows[e, sl] + attrv[e, sl],
                                               0.0) * s_ref[sl])
                return 0

            lax.fori_loop(0, BATCH, _edge, 0)

        @pl.when((d1 == 1) & (dl > 1))
        def _mixed():
            distv = bufs[p][1]

            def _edge(e, _):
                esplat = jnp.full((16,), e, jnp.int32)
                dsplat = plsc.load_gather(distv, [esplat])
                m = dsplat == 1
                for jj in range(DIM // 16):
                    sl = pl.ds(16 * jj, 16)
                    u = rows[e, sl]
                    t = jnp.maximum(u + attrv[e, sl], 0.0) * s_ref[sl]
                    rows[e, sl] = jnp.where(m, t, u)
                return 0

            lax.fori_loop(0, BATCH, _edge, 0)

        pltpu.make_async_copy(dst_hbm.at[pl.ds(off, BATCH)], dstv,
                              bufs[p][8]).wait()
        pltpu.async_copy(rows, acc.at[dstv], bufs[p][11], add=True)

    def scat_wait(p):
        pltpu.make_async_copy(bufs[p][4], acc.at[bufs[p][2]],
                              bufs[p][11]).wait()


    idx_start(0, 0)
    idx_start(1, 1)
    dst_start(0, 0)
    launch(0, 0)

    def _pair(i, _):
        k0 = 2 * i

        @pl.when(k0 > 0)
        def _():
            scat_wait(1)
        dst_start(k0 + 1, 1)
        launch(k0 + 1, 1)
        process(k0, 0)
        idx_start(k0 + 2, 0)
        process(k0 + 1, 1)
        scat_wait(0)
        dst_start(k0 + 2, 0)
        launch(k0 + 2, 0)

        @pl.when(k0 + 3 < NUM_BLOCKS)
        def _():
            idx_start(k0 + 3, 1)
        return 0

    lax.fori_loop(0, (NUM_BLOCKS - 1) // 2, _pair, 0)

    scat_wait(1)
    process(NUM_BLOCKS - 1, 0)
    scat_wait(0)
    plsc.subcore_barrier()

    def _read_chunk(t, _):
        c = sub + NUM_SUBCORES * t

        @pl.when(c < NUM_ROW_CHUNKS)
        def _():
            r0 = pl.multiple_of(c * ROW_CHUNK, 8)
            pltpu.sync_copy(acc.at[pl.ds(r0, ROW_CHUNK)], rows_a)
            pltpu.sync_copy(rows_a, out_hbm.at[core, pl.ds(r0, ROW_CHUNK)])
        return 0

    lax.fori_loop(0, pl.cdiv(NUM_ROW_CHUNKS, NUM_SUBCORES), _read_chunk, 0)


def _sc_edges(u4, src, dst, dist, edge_attr, eps_flat):
    mesh = plsc.VectorSubcoreMesh(core_axis_name="c", subcore_axis_name="s")
    ivec = pltpu.VMEM((BATCH,), jnp.int32)
    fmat = pltpu.VMEM((BATCH, DIM), jnp.float32)
    sem = pltpu.SemaphoreType.DMA
    return pl.kernel(
        _sc_edges_body,
        out_type=jax.ShapeDtypeStruct((NUM_CORES, N, DIM), jnp.float32),
        mesh=mesh,
        compiler_params=pltpu.CompilerParams(needs_layout_passes=False),
        scratch_types=[
            ivec, ivec,
            ivec, ivec,
            ivec, ivec,
            ivec, ivec,
            fmat, fmat,
            fmat, fmat,
            pltpu.VMEM((32,), jnp.int32),
            pltpu.VMEM((32,), jnp.int32),
            pltpu.VMEM(((K + 1) * DIM,), jnp.float32),
            pltpu.VMEM((DIM,), jnp.float32),
            pltpu.VMEM_SHARED((N, DIM), jnp.float32),
            sem, sem, sem, sem, sem, sem, sem, sem, sem, sem,
        ],
    )(u4, src, dst, dist, edge_attr, eps_flat)


def _mlp_head_body(x_ref, part_ref, eps_ref, w1_ref, b1_ref, gamma_ref,
                   beta_ref, w2_ref, b2_ref, out_ref):
    result = ((1.0 + eps_ref[0])[None, :] * x_ref[...]
              + part_ref[0] + part_ref[1])
    h = jnp.dot(result, w1_ref[...],
                preferred_element_type=jnp.float32) + b1_ref[...][None, :]
    mean = jnp.mean(h, axis=0)
    var = jnp.mean((h - mean[None, :]) ** 2, axis=0)
    h = (h - mean[None, :]) / jnp.sqrt(var + 1e-5)
    h = h * gamma_ref[...][None, :] + beta_ref[...][None, :]
    h = jnp.maximum(h, 0.0)
    out_ref[...] = jnp.dot(h, w2_ref[...],
                           preferred_element_type=jnp.float32) \
        + b2_ref[...][None, :]


def _mlp_head(x, part, eps, W1, b1, gamma, beta, W2, b2):
    return pl.pallas_call(
        _mlp_head_body,
        out_shape=jax.ShapeDtypeStruct((N, DIM), jnp.float32),
    )(x, part, eps, W1, b1, gamma, beta, W2, b2)


def kernel(x, multihop_edge_index, distance, edge_attr, eps,
           W1, b1, gamma, beta, W2, b2):
    src = multihop_edge_index[0]
    dst = multihop_edge_index[1]
    u4 = _build_tables(x, eps).reshape(K * N, DIM)
    part = _sc_edges(u4, src, dst, distance, edge_attr, eps.reshape(-1))
    return _mlp_head(x, part, eps, W1, b1, gamma, beta, W2, b2)


# speedup vs baseline: 16.0362x; 1.0572x over previous
import functools

import jax
import jax.numpy as jnp
from jax import lax
from jax.experimental import pallas as pl
from jax.experimental.pallas import tpu as pltpu
from jax.experimental.pallas import tpu_sc as plsc

N = 10000
E = 320000
DIM = 128
K = 4

NUM_CORES = 2
NUM_SUBCORES = 16
NUM_TILES = NUM_CORES * NUM_SUBCORES
EDGES_PER_TILE = E // NUM_TILES
BATCH = 80
NUM_BLOCKS = EDGES_PER_TILE // BATCH
ROW_CHUNK = 80
NUM_ROW_CHUNKS = N // ROW_CHUNK


def _build_tables_body(x_ref, eps_ref, out_ref):
    xb = x_ref[...]
    rb = jnp.maximum(xb, 0.0)
    out_ref[0] = xb
    for j in range(1, K):
        out_ref[j] = rb * (1.0 + eps_ref[j + 1])[None, :]


def _build_tables(x, eps):
    blk = 1000
    return pl.pallas_call(
        _build_tables_body,
        grid=(N // blk,),
        in_specs=[
            pl.BlockSpec((blk, DIM), lambda i: (i, 0)),
            pl.BlockSpec((K + 1, DIM), lambda i: (0, 0)),
        ],
        out_specs=pl.BlockSpec((K, blk, DIM), lambda i: (0, i, 0)),
        out_shape=jax.ShapeDtypeStruct((K, N, DIM), jnp.float32),
    )(x, eps)


def _sc_edges_body(u_hbm, src_hbm, dst_hbm, dist_hbm, attr_hbm, eps_hbm,
                   out_hbm,
                   srcv_a, srcv_b, distv_a, distv_b, dstv_a, dstv_b,
                   gidx_a, gidx_b, rows_a, rows_b, attrv_a, attrv_b,
                   dchk_a, dchk_b, epsv, s_ref, acc,
                   isem_a, isem_b, dsem_a, dsem_b, gsem_a, gsem_b,
                   asem_a, asem_b, ssem_a, ssem_b):
    core = lax.axis_index("c")
    sub = lax.axis_index("s")
    tile = core * NUM_SUBCORES + sub

    pltpu.sync_copy(eps_hbm, epsv)
    for j in range(DIM // 16):
        s_ref[pl.ds(16 * j, 16)] = epsv[pl.ds(DIM + 16 * j, 16)] + 1.0

    zeros16 = jnp.zeros((16,), jnp.float32)

    def _zero_row(r, _):
        for j in range(DIM // 16):
            rows_a[r, pl.ds(16 * j, 16)] = zeros16
        return 0

    lax.fori_loop(0, ROW_CHUNK, _zero_row, 0)

    def _zero_chunk(t, _):
        c = sub + NUM_SUBCORES * t

        @pl.when(c < NUM_ROW_CHUNKS)
        def _():
            r0 = pl.multiple_of(c * ROW_CHUNK, 8)
            pltpu.sync_copy(rows_a, acc.at[pl.ds(r0, ROW_CHUNK)])
        return 0

    lax.fori_loop(0, pl.cdiv(NUM_ROW_CHUNKS, NUM_SUBCORES), _zero_chunk, 0)
    plsc.subcore_barrier()

    def _off(j):
        return pl.multiple_of(tile * BATCH + j * (NUM_TILES * BATCH), 8)

    bufs = (
        (srcv_a, distv_a, dstv_a, gidx_a, rows_a, attrv_a, dchk_a,
         isem_a, dsem_a, gsem_a, asem_a, ssem_a),
        (srcv_b, distv_b, dstv_b, gidx_b, rows_b, attrv_b, dchk_b,
         isem_b, dsem_b, gsem_b, asem_b, ssem_b),
    )

    def idx_start(j, p):
        srcv, distv = bufs[p][0], bufs[p][1]
        off = _off(j)
        pltpu.async_copy(src_hbm.at[pl.ds(off, BATCH)], srcv, bufs[p][7])
        pltpu.async_copy(dist_hbm.at[pl.ds(off, BATCH)], distv, bufs[p][7])

    def dst_start(j, p):
        off = _off(j)
        pltpu.async_copy(dst_hbm.at[pl.ds(off, BATCH)], bufs[p][2],
                         bufs[p][8])

    def launch(j, p):
        srcv, distv, _, gidx, rows, attrv, dchk = bufs[p][:7]
        off = _off(j)
        pltpu.make_async_copy(src_hbm.at[pl.ds(off, BATCH)], srcv,
                              bufs[p][7]).wait()
        pltpu.make_async_copy(dist_hbm.at[pl.ds(off, BATCH)], distv,
                              bufs[p][7]).wait()
        for i in range(BATCH // 16):
            sl = pl.ds(16 * i, 16)
            gidx[sl] = (distv[sl] - 1) * N + srcv[sl]
        dchk[pl.ds(0, 16)] = distv[pl.ds(0, 16)]
        dchk[pl.ds(16, 16)] = distv[pl.ds(BATCH - 16, 16)]
        d1 = jnp.min(distv[pl.ds(0, 16)])

        @pl.when(d1 == 1)
        def _():
            pltpu.async_copy(attr_hbm.at[pl.ds(off, BATCH)], attrv,
                             bufs[p][10])

        pltpu.async_copy(u_hbm.at[gidx], rows, bufs[p][9])

    def process(j, p):
        _, _, dstv, gidx, rows, attrv, dchk = bufs[p][:7]
        off = _off(j)
        pltpu.make_async_copy(u_hbm.at[gidx], rows, bufs[p][9]).wait()
        d1 = jnp.min(dchk[pl.ds(0, 16)])
        dl = jnp.max(dchk[pl.ds(16, 16)])

        @pl.when(d1 == 1)
        def _():
            pltpu.make_async_copy(attr_hbm.at[pl.ds(off, BATCH)], attrv,
                                  bufs[p][10]).wait()

        @pl.when(dl == 1)
        def _pure_hop1():
            def _edge(e, _):
                for jj in range(DIM // 16):
                    sl = pl.ds(16 * jj, 16)
                    rows[e, sl] = (jnp.maximum(rows[e, sl] + attrv[e, sl],
                                               0.0) * s_ref[sl])
                return 0

            lax.fori_loop(0, BATCH, _edge, 0)

        @pl.when((d1 == 1) & (dl > 1))
        def _mixed():
            distv = bufs[p][1]

            def _edge(e, _):
                esplat = jnp.full((16,), e, jnp.int32)
                dsplat = plsc.load_gather(distv, [esplat])
                m = dsplat == 1
                for jj in range(DIM // 16):
                    sl = pl.ds(16 * jj, 16)
                    u = rows[e, sl]
                    t = jnp.maximum(u + attrv[e, sl], 0.0) * s_ref[sl]
                    rows[e, sl] = jnp.where(m, t, u)
                return 0

            lax.fori_loop(0, BATCH, _edge, 0)

        pltpu.make_async_copy(dst_hbm.at[pl.ds(off, BATCH)], dstv,
                              bufs[p][8]).wait()
        pltpu.async_copy(rows, acc.at[dstv], bufs[p][11], add=True)

    def scat_wait(p):
        pltpu.make_async_copy(bufs[p][4], acc.at[bufs[p][2]],
                              bufs[p][11]).wait()


    idx_start(0, 0)
    idx_start(1, 1)
    dst_start(0, 0)
    launch(0, 0)

    def _pair(i, _):
        k0 = 2 * i

        @pl.when(k0 > 0)
        def _():
            scat_wait(1)
        dst_start(k0 + 1, 1)
        launch(k0 + 1, 1)
        process(k0, 0)
        idx_start(k0 + 2, 0)

        scat_wait(0)
        dst_start(k0 + 2, 0)
        launch(k0 + 2, 0)
        process(k0 + 1, 1)

        @pl.when(k0 + 3 < NUM_BLOCKS)
        def _():
            idx_start(k0 + 3, 1)
        return 0

    lax.fori_loop(0, (NUM_BLOCKS - 1) // 2, _pair, 0)

    scat_wait(1)
    process(NUM_BLOCKS - 1, 0)
    scat_wait(0)
    plsc.subcore_barrier()

    rbufs = (rows_a, rows_b)
    rsems = (gsem_a, gsem_b)

    def _ro_pull(t, q):
        c = sub + NUM_SUBCORES * t

        @pl.when(c < NUM_ROW_CHUNKS)
        def _():
            r0 = pl.multiple_of(c * ROW_CHUNK, 8)
            pltpu.async_copy(acc.at[pl.ds(r0, ROW_CHUNK)], rbufs[q],
                             rsems[q])

    def _ro_push(t, q):
        c = sub + NUM_SUBCORES * t

        @pl.when(c < NUM_ROW_CHUNKS)
        def _():
            r0 = pl.multiple_of(c * ROW_CHUNK, 8)
            pltpu.make_async_copy(acc.at[pl.ds(r0, ROW_CHUNK)], rbufs[q],
                                  rsems[q]).wait()
            pltpu.sync_copy(rbufs[q], out_hbm.at[core, pl.ds(r0, ROW_CHUNK)])

    _ro_pull(0, 0)

    def _ro_pair(t2, _):
        t0 = 2 * t2
        _ro_pull(t0 + 1, 1)
        _ro_push(t0, 0)
        _ro_pull(t0 + 2, 0)
        _ro_push(t0 + 1, 1)
        return 0

    lax.fori_loop(0, pl.cdiv(NUM_ROW_CHUNKS, NUM_SUBCORES) // 2, _ro_pair, 0)


def _sc_edges(u4, src, dst, dist, edge_attr, eps_flat):
    mesh = plsc.VectorSubcoreMesh(core_axis_name="c", subcore_axis_name="s")
    ivec = pltpu.VMEM((BATCH,), jnp.int32)
    fmat = pltpu.VMEM((BATCH, DIM), jnp.float32)
    sem = pltpu.SemaphoreType.DMA
    return pl.kernel(
        _sc_edges_body,
        out_type=jax.ShapeDtypeStruct((NUM_CORES, N, DIM), jnp.float32),
        mesh=mesh,
        compiler_params=pltpu.CompilerParams(needs_layout_passes=False),
        scratch_types=[
            ivec, ivec,
            ivec, ivec,
            ivec, ivec,
            ivec, ivec,
            fmat, fmat,
            fmat, fmat,
            pltpu.VMEM((32,), jnp.int32),
            pltpu.VMEM((32,), jnp.int32),
            pltpu.VMEM(((K + 1) * DIM,), jnp.float32),
            pltpu.VMEM((DIM,), jnp.float32),
            pltpu.VMEM_SHARED((N, DIM), jnp.float32),
            sem, sem, sem, sem, sem, sem, sem, sem, sem, sem,
        ],
    )(u4, src, dst, dist, edge_attr, eps_flat)


def _mlp_head_body(x_ref, part_ref, eps_ref, w1_ref, b1_ref, gamma_ref,
                   beta_ref, w2_ref, b2_ref, out_ref):
    result = ((1.0 + eps_ref[0])[None, :] * x_ref[...]
              + part_ref[0] + part_ref[1])
    h = jnp.dot(result, w1_ref[...],
                preferred_element_type=jnp.float32) + b1_ref[...][None, :]
    mean = jnp.mean(h, axis=0)
    var = jnp.mean((h - mean[None, :]) ** 2, axis=0)
    h = (h - mean[None, :]) / jnp.sqrt(var + 1e-5)
    h = h * gamma_ref[...][None, :] + beta_ref[...][None, :]
    h = jnp.maximum(h, 0.0)
    out_ref[...] = jnp.dot(h, w2_ref[...],
                           preferred_element_type=jnp.float32) \
        + b2_ref[...][None, :]


def _mlp_head(x, part, eps, W1, b1, gamma, beta, W2, b2):
    return pl.pallas_call(
        _mlp_head_body,
        out_shape=jax.ShapeDtypeStruct((N, DIM), jnp.float32),
    )(x, part, eps, W1, b1, gamma, beta, W2, b2)


def kernel(x, multihop_edge_index, distance, edge_attr, eps,
           W1, b1, gamma, beta, W2, b2):
    u4 = _build_tables(x, eps).reshape(K * N, DIM)
    part = _sc_edges(u4, multihop_edge_index[0], multihop_edge_index[1],
                     distance, edge_attr, eps.reshape(-1))
    return _mlp_head(x, part, eps, W1, b1, gamma, beta, W2, b2)
